# baseline (speedup 1.0000x reference)
import dataclasses
import functools

import jax
import jax.numpy as jnp
from jax import lax
from jax.experimental import pallas as pl
from jax.experimental.pallas import tpu as pltpu
from jax.experimental.pallas import tpu_sc as plsc

N = 100000
D = 128
NUM_SEG = 2048

SC_CORES = 2
SC_SUBCORES = 16
LANES = 16

CHUNK_A = 6400
CHUNK_B = N - 15 * CHUNK_A
ACC_LEN = NUM_SEG + 8
COLS = NUM_SEG // SC_SUBCORES

FILL_A = 6528
FILL_B = (N - NUM_SEG) - 15 * FILL_A

TC_BLOCK = 10000
NB = N // TC_BLOCK
TC_SUB = 5


def _tc_body(g_ref, hn_ref, w1_ref, b1_ref, w2_ref, b2_ref,
             ho_ref, co_ref, gate_ref):
    w1 = w1_ref[...].astype(jnp.bfloat16)
    w2 = w2_ref[...].astype(jnp.bfloat16)
    H = TC_BLOCK // TC_SUB
    gates = []
    for p in range(TC_SUB):
        rows = pl.ds(p * H, H)
        x = jnp.dot(g_ref[rows, :].astype(jnp.bfloat16), w1,
                    preferred_element_type=jnp.float32)
        x = jnp.maximum(x + b1_ref[...], 0.0).astype(jnp.bfloat16)
        logit = jnp.dot(x, w2,
                        preferred_element_type=jnp.float32)
        gate = jax.nn.sigmoid(logit + b2_ref[...])
        gates.append(gate)
        hn = hn_ref[rows, :]
        ho = gate * hn
        ho_ref[rows, :] = ho
        co_ref[rows, :] = hn - ho
    gate_row = lax.transpose(jnp.concatenate(gates, axis=0), (1, 0))
    gate_ref[...] = gate_row.reshape(1, 1, TC_BLOCK)


def _tc_fused(g, h_node, W1, b1, W2, b2):
    return pl.pallas_call(
        _tc_body,
        grid=(NB,),
        in_specs=[
            pl.BlockSpec((TC_BLOCK, D), lambda i: (i, 0)),
            pl.BlockSpec((TC_BLOCK, D), lambda i: (i, 0)),
            pl.BlockSpec((D, D), lambda i: (0, 0)),
            pl.BlockSpec((1, D), lambda i: (0, 0)),
            pl.BlockSpec((D, 1), lambda i: (0, 0)),
            pl.BlockSpec((1, 1), lambda i: (0, 0)),
        ],
        out_specs=[
            pl.BlockSpec((TC_BLOCK, D), lambda i: (i, 0)),
            pl.BlockSpec((TC_BLOCK, D), lambda i: (i, 0)),
            pl.BlockSpec((1, 1, TC_BLOCK), lambda i: (i, 0, 0)),
        ],
        out_shape=[
            jax.ShapeDtypeStruct((N, D), jnp.float32),
            jax.ShapeDtypeStruct((N, D), jnp.float32),
            jax.ShapeDtypeStruct((NB, 1, TC_BLOCK), jnp.float32),
        ],
    )(g, h_node, W1, b1.reshape(1, D), W2, b2.reshape(1, 1))


def _accumulate(gate_v, idx_v, acc_v, length, sgn, off):
    @pl.loop(0, length, step=LANES, unroll=8)
    def _(i):
        v = gate_v[pl.ds(i, LANES)]
        ii = idx_v[pl.ds(i, LANES)]
        plsc.addupdate_scatter(acc_v, [ii], sgn * v + off)


def _sc_body(gate_hbm, idx_hbm, r_hbm, env_hbm,
             gate_v, idx_v, acc_v, tmp_v, res_v, fill_v, shared):
    cid = lax.axis_index("c")
    sid = lax.axis_index("s")

    @pl.loop(0, ACC_LEN, step=LANES)
    def _(i):
        acc_v[pl.ds(i, LANES)] = jnp.zeros((LANES,), jnp.float32)

    cid_f = cid.astype(jnp.float32)
    sgn = 1.0 - 2.0 * cid_f
    off = cid_f

    @pl.when(sid < SC_SUBCORES - 1)
    def _():
        base = sid * CHUNK_A
        pltpu.sync_copy(gate_hbm.at[pl.ds(base, CHUNK_A)],
                        gate_v.at[pl.ds(0, CHUNK_A)])
        pltpu.sync_copy(idx_hbm.at[pl.ds(base, CHUNK_A)],
                        idx_v.at[pl.ds(0, CHUNK_A)])
        _accumulate(gate_v, idx_v, acc_v, CHUNK_A, sgn, off)

    @pl.when(sid == SC_SUBCORES - 1)
    def _():
        base = 15 * CHUNK_A
        pltpu.sync_copy(gate_hbm.at[pl.ds(base, CHUNK_B)],
                        gate_v.at[pl.ds(0, CHUNK_B)])
        pltpu.sync_copy(idx_hbm.at[pl.ds(base, CHUNK_B)],
                        idx_v.at[pl.ds(0, CHUNK_B)])
        _accumulate(gate_v, idx_v, acc_v, CHUNK_B, sgn, off)

    pltpu.sync_copy(acc_v.at[pl.ds(0, NUM_SEG)], shared.at[sid])

    @pl.loop(0, FILL_A, step=LANES)
    def _(i):
        fill_v[pl.ds(i, LANES)] = jnp.full((LANES,), 1e-8, jnp.float32)

    @pl.when(sid < SC_SUBCORES - 1)
    def _():
        @pl.when(cid == 0)
        def _():
            pltpu.sync_copy(fill_v, r_hbm.at[pl.ds(NUM_SEG + sid * FILL_A,
                                                   FILL_A)])
        @pl.when(cid == 1)
        def _():
            pltpu.sync_copy(fill_v, env_hbm.at[pl.ds(NUM_SEG + sid * FILL_A,
                                                     FILL_A)])

    @pl.when(sid == SC_SUBCORES - 1)
    def _():
        @pl.when(cid == 0)
        def _():
            pltpu.sync_copy(fill_v.at[pl.ds(0, FILL_B)],
                            r_hbm.at[pl.ds(N - FILL_B, FILL_B)])
        @pl.when(cid == 1)
        def _():
            pltpu.sync_copy(fill_v.at[pl.ds(0, FILL_B)],
                            env_hbm.at[pl.ds(N - FILL_B, FILL_B)])

    plsc.subcore_barrier()

    col = sid * COLS

    @pl.loop(0, COLS, step=LANES)
    def _(i):
        res_v[pl.ds(i, LANES)] = jnp.full((LANES,), 1e-8, jnp.float32)

    for r in range(SC_SUBCORES):
        pltpu.sync_copy(shared.at[r, pl.ds(col, COLS)], tmp_v)

        @pl.loop(0, COLS, step=LANES)
        def _(i):
            res_v[pl.ds(i, LANES)] += tmp_v[pl.ds(i, LANES)]

    @pl.when(cid == 0)
    def _():
        pltpu.sync_copy(res_v, r_hbm.at[pl.ds(col, COLS)])

    @pl.when(cid == 1)
    def _():
        pltpu.sync_copy(res_v, env_hbm.at[pl.ds(col, COLS)])


def _sc_segment_sums(gate_flat, idx):
    mesh = plsc.VectorSubcoreMesh(core_axis_name="c", subcore_axis_name="s")
    cp = pltpu.CompilerParams()
    if "needs_layout_passes" in pltpu.CompilerParams.__dataclass_fields__:
        cp = dataclasses.replace(cp, needs_layout_passes=False)
    k = pl.kernel(
        _sc_body,
        out_type=[jax.ShapeDtypeStruct((N,), jnp.float32),
                  jax.ShapeDtypeStruct((N,), jnp.float32)],
        mesh=mesh,
        scratch_types=[
            pltpu.VMEM((CHUNK_A,), jnp.float32),
            pltpu.VMEM((CHUNK_A,), jnp.int32),
            pltpu.VMEM((ACC_LEN,), jnp.float32),
            pltpu.VMEM((COLS,), jnp.float32),
            pltpu.VMEM((COLS,), jnp.float32),
            pltpu.VMEM((FILL_A,), jnp.float32),
            pltpu.VMEM_SHARED((SC_SUBCORES, NUM_SEG), jnp.float32),
        ],
        compiler_params=cp,
    )
    return k(gate_flat, idx)


def kernel(g, h, h_node, W1, b1, W2, b2):
    h_out, c_out, gate_rows = _tc_fused(g, h_node, W1, b1, W2, b2)

    r_flat, env_flat = _sc_segment_sums(gate_rows.reshape(N),
                                        h.astype(jnp.int32))

    return (h_out, c_out, r_flat.reshape(N, 1), env_flat.reshape(N, 1))


# speedup vs baseline: 1.0155x; 1.0155x over previous
import dataclasses
import functools

import jax
import jax.numpy as jnp
from jax import lax
from jax.experimental import pallas as pl
from jax.experimental.pallas import tpu as pltpu
from jax.experimental.pallas import tpu_sc as plsc

N = 100000
D = 128
NUM_SEG = 2048

SC_CORES = 2
SC_SUBCORES = 16
LANES = 16

CHUNK_A = 6400
CHUNK_B = N - 15 * CHUNK_A
ACC_LEN = NUM_SEG + 8
COLS = NUM_SEG // SC_SUBCORES

FILL_A = 6528
FILL_B = (N - NUM_SEG) - 15 * FILL_A

TC_BLOCK = 10000
NB = N // TC_BLOCK
TC_SUB = 5


def _tc_body(g_ref, hn_ref, w1_ref, b1_ref, w2_ref, b2_ref,
             ho_ref, co_ref, gate_ref):
    w1 = w1_ref[...].astype(jnp.bfloat16)
    w2 = w2_ref[...].astype(jnp.bfloat16)
    H = TC_BLOCK // TC_SUB
    gates = []
    for p in range(TC_SUB):
        rows = pl.ds(p * H, H)
        x = jnp.dot(g_ref[rows, :].astype(jnp.bfloat16), w1,
                    preferred_element_type=jnp.float32)
        x = jnp.maximum(x + b1_ref[...], 0.0).astype(jnp.bfloat16)
        logit = jnp.dot(x, w2,
                        preferred_element_type=jnp.float32)
        gate = jax.nn.sigmoid(logit + b2_ref[...])
        gates.append(gate)
        hn = hn_ref[rows, :]
        ho = gate * hn
        ho_ref[rows, :] = ho
        co_ref[rows, :] = hn - ho
    gate_row = lax.transpose(jnp.concatenate(gates, axis=0), (1, 0))
    gate_ref[...] = gate_row.reshape(1, 1, TC_BLOCK)


def _tc_fused(g, h_node, W1, b1, W2, b2):
    return pl.pallas_call(
        _tc_body,
        grid=(NB,),
        in_specs=[
            pl.BlockSpec((TC_BLOCK, D), lambda i: (i, 0)),
            pl.BlockSpec((TC_BLOCK, D), lambda i: (i, 0)),
            pl.BlockSpec((D, D), lambda i: (0, 0)),
            pl.BlockSpec((1, D), lambda i: (0, 0)),
            pl.BlockSpec((D, 1), lambda i: (0, 0)),
            pl.BlockSpec((1, 1), lambda i: (0, 0)),
        ],
        out_specs=[
            pl.BlockSpec((TC_BLOCK, D), lambda i: (i, 0)),
            pl.BlockSpec((TC_BLOCK, D), lambda i: (i, 0)),
            pl.BlockSpec((1, 1, TC_BLOCK), lambda i: (i, 0, 0)),
        ],
        out_shape=[
            jax.ShapeDtypeStruct((N, D), jnp.float32),
            jax.ShapeDtypeStruct((N, D), jnp.float32),
            jax.ShapeDtypeStruct((NB, 1, TC_BLOCK), jnp.float32),
        ],
    )(g, h_node, W1, b1.reshape(1, D), W2, b2.reshape(1, 1))


def _accumulate(gate_v, idx_v, acc_v, length, sgn, off):
    stride = length // LANES
    base = lax.iota(jnp.int32, LANES) * stride

    @pl.loop(0, stride, step=1, unroll=8)
    def _(i):
        sel = base + i
        v = plsc.load_gather(gate_v, [sel])
        ii = plsc.load_gather(idx_v, [sel])
        plsc.addupdate_scatter(acc_v, [ii], sgn * v + off)


def _sc_body(gate_hbm, idx_hbm, r_hbm, env_hbm,
             gate_v, idx_v, acc_v, tmp_v, res_v, fill_v, shared):
    cid = lax.axis_index("c")
    sid = lax.axis_index("s")

    @pl.loop(0, ACC_LEN, step=LANES)
    def _(i):
        acc_v[pl.ds(i, LANES)] = jnp.zeros((LANES,), jnp.float32)

    cid_f = cid.astype(jnp.float32)
    sgn = 1.0 - 2.0 * cid_f
    off = cid_f

    @pl.when(sid < SC_SUBCORES - 1)
    def _():
        base = sid * CHUNK_A
        pltpu.sync_copy(gate_hbm.at[pl.ds(base, CHUNK_A)],
                        gate_v.at[pl.ds(0, CHUNK_A)])
        pltpu.sync_copy(idx_hbm.at[pl.ds(base, CHUNK_A)],
                        idx_v.at[pl.ds(0, CHUNK_A)])
        _accumulate(gate_v, idx_v, acc_v, CHUNK_A, sgn, off)

    @pl.when(sid == SC_SUBCORES - 1)
    def _():
        base = 15 * CHUNK_A
        pltpu.sync_copy(gate_hbm.at[pl.ds(base, CHUNK_B)],
                        gate_v.at[pl.ds(0, CHUNK_B)])
        pltpu.sync_copy(idx_hbm.at[pl.ds(base, CHUNK_B)],
                        idx_v.at[pl.ds(0, CHUNK_B)])
        _accumulate(gate_v, idx_v, acc_v, CHUNK_B, sgn, off)

    pltpu.sync_copy(acc_v.at[pl.ds(0, NUM_SEG)], shared.at[sid])

    @pl.loop(0, FILL_A, step=LANES)
    def _(i):
        fill_v[pl.ds(i, LANES)] = jnp.full((LANES,), 1e-8, jnp.float32)

    @pl.when(sid < SC_SUBCORES - 1)
    def _():
        @pl.when(cid == 0)
        def _():
            pltpu.sync_copy(fill_v, r_hbm.at[pl.ds(NUM_SEG + sid * FILL_A,
                                                   FILL_A)])
        @pl.when(cid == 1)
        def _():
            pltpu.sync_copy(fill_v, env_hbm.at[pl.ds(NUM_SEG + sid * FILL_A,
                                                     FILL_A)])

    @pl.when(sid == SC_SUBCORES - 1)
    def _():
        @pl.when(cid == 0)
        def _():
            pltpu.sync_copy(fill_v.at[pl.ds(0, FILL_B)],
                            r_hbm.at[pl.ds(N - FILL_B, FILL_B)])
        @pl.when(cid == 1)
        def _():
            pltpu.sync_copy(fill_v.at[pl.ds(0, FILL_B)],
                            env_hbm.at[pl.ds(N - FILL_B, FILL_B)])

    plsc.subcore_barrier()

    col = sid * COLS

    @pl.loop(0, COLS, step=LANES)
    def _(i):
        res_v[pl.ds(i, LANES)] = jnp.full((LANES,), 1e-8, jnp.float32)

    for r in range(SC_SUBCORES):
        pltpu.sync_copy(shared.at[r, pl.ds(col, COLS)], tmp_v)

        @pl.loop(0, COLS, step=LANES)
        def _(i):
            res_v[pl.ds(i, LANES)] += tmp_v[pl.ds(i, LANES)]

    @pl.when(cid == 0)
    def _():
        pltpu.sync_copy(res_v, r_hbm.at[pl.ds(col, COLS)])

    @pl.when(cid == 1)
    def _():
        pltpu.sync_copy(res_v, env_hbm.at[pl.ds(col, COLS)])


def _sc_segment_sums(gate_flat, idx):
    mesh = plsc.VectorSubcoreMesh(core_axis_name="c", subcore_axis_name="s")
    cp = pltpu.CompilerParams()
    if "needs_layout_passes" in pltpu.CompilerParams.__dataclass_fields__:
        cp = dataclasses.replace(cp, needs_layout_passes=False)
    k = pl.kernel(
        _sc_body,
        out_type=[jax.ShapeDtypeStruct((N,), jnp.float32),
                  jax.ShapeDtypeStruct((N,), jnp.float32)],
        mesh=mesh,
        scratch_types=[
            pltpu.VMEM((CHUNK_A,), jnp.float32),
            pltpu.VMEM((CHUNK_A,), jnp.int32),
            pltpu.VMEM((ACC_LEN,), jnp.float32),
            pltpu.VMEM((COLS,), jnp.float32),
            pltpu.VMEM((COLS,), jnp.float32),
            pltpu.VMEM((FILL_A,), jnp.float32),
            pltpu.VMEM_SHARED((SC_SUBCORES, NUM_SEG), jnp.float32),
        ],
        compiler_params=cp,
    )
    return k(gate_flat, idx)


def kernel(g, h, h_node, W1, b1, W2, b2):
    h_out, c_out, gate_rows = _tc_fused(g, h_node, W1, b1, W2, b2)

    r_flat, env_flat = _sc_segment_sums(gate_rows.reshape(N),
                                        h.astype(jnp.int32))

    return (h_out, c_out, r_flat.reshape(N, 1), env_flat.reshape(N, 1))
